# baseline (speedup 1.0000x reference)
import functools

import jax
import jax.numpy as jnp
from jax import lax
from jax.experimental import pallas as pl
from jax.experimental.pallas import tpu as pltpu
from jax.experimental.pallas import tpu_sc as plsc

T = 32768
D = 768
E = 8
K = 2

NC = 2
NS = 16
L = 16
NW = NC * NS
TPW = T // NW
GROUPS = TPW // L

BLK = 2048


def _logits_body(x_ref, wt_ref, b_ref, out_ref):
    out_ref[...] = (
        jnp.dot(x_ref[...], wt_ref[...], preferred_element_type=jnp.float32)
        + b_ref[...]
    )


_logits_call = pl.pallas_call(
    _logits_body,
    grid=(T // BLK,),
    in_specs=[
        pl.BlockSpec((BLK, D), lambda i: (i, 0)),
        pl.BlockSpec((D, E), lambda i: (0, 0)),
        pl.BlockSpec((1, E), lambda i: (0, 0)),
    ],
    out_specs=pl.BlockSpec((BLK, E), lambda i: (i, 0)),
    out_shape=jax.ShapeDtypeStruct((T, E), jnp.float32),
    compiler_params=pltpu.CompilerParams(
        dimension_semantics=("arbitrary",),
    ),
)


@functools.partial(
    pl.kernel,
    out_type=(
        jax.ShapeDtypeStruct((T * K,), jnp.int32),
        jax.ShapeDtypeStruct((T * K,), jnp.float32),
    ),
    mesh=plsc.VectorSubcoreMesh(core_axis_name="c", subcore_axis_name="s"),
    compiler_params=pltpu.CompilerParams(needs_layout_passes=False),
    scratch_types=[
        pltpu.VMEM((TPW * E,), jnp.float32),
        pltpu.VMEM((TPW * K,), jnp.int32),
        pltpu.VMEM((TPW * K,), jnp.float32),
    ],
)
def _route(logits_hbm, idx_hbm, gate_hbm, lg_v, idx_v, gate_v):
    wid = lax.axis_index("s") * NC + lax.axis_index("c")
    base = wid * TPW
    pltpu.sync_copy(logits_hbm.at[pl.ds(base * E, TPW * E)], lg_v)

    def body(g, carry):
        row = g * L + lax.iota(jnp.int32, L)
        ls = [
            plsc.load_gather(lg_v, [row * E + jnp.full((L,), e, jnp.int32)])
            for e in range(E)
        ]
        v1 = ls[0]
        i1 = jnp.zeros((L,), jnp.int32)
        v2 = jnp.full((L,), -jnp.inf, jnp.float32)
        i2 = jnp.zeros((L,), jnp.int32)
        for e in range(1, E):
            le = ls[e]
            ee = jnp.full((L,), e, jnp.int32)
            gt1 = le > v1
            gt2 = le > v2
            v2 = jnp.where(gt1, v1, jnp.where(gt2, le, v2))
            i2 = jnp.where(gt1, i1, jnp.where(gt2, ee, i2))
            v1 = jnp.where(gt1, le, v1)
            i1 = jnp.where(gt1, ee, i1)
        s = jnp.exp(ls[0] - v1)
        for e in range(1, E):
            s = s + jnp.exp(ls[e] - v1)
        g1 = 1.0 / s
        g2 = jnp.exp(v2 - v1) / s
        pos = row * K
        one = jnp.full((L,), 1, jnp.int32)
        plsc.store_scatter(idx_v, [pos], i1)
        plsc.store_scatter(idx_v, [pos + one], i2)
        plsc.store_scatter(gate_v, [pos], g1)
        plsc.store_scatter(gate_v, [pos + one], g2)
        return carry

    lax.fori_loop(0, GROUPS, body, 0)
    pltpu.sync_copy(idx_v, idx_hbm.at[pl.ds(base * K, TPW * K)])
    pltpu.sync_copy(gate_v, gate_hbm.at[pl.ds(base * K, TPW * K)])


def kernel(x, W, b):
    logits = _logits_call(x, W.T, b.reshape(1, E))
    idx_flat, gate_flat = _route(logits.reshape(T * E))
    return idx_flat.reshape(T, K), gate_flat.reshape(T, K)


# speedup vs baseline: 2.1685x; 2.1685x over previous
import functools

import jax
import jax.numpy as jnp
from jax import lax
from jax.experimental import pallas as pl
from jax.experimental.pallas import tpu as pltpu
from jax.experimental.pallas import tpu_sc as plsc

T = 32768
D = 768
E = 8
K = 2

NC = 2
NS = 16
L = 16
NW = NC * NS
TPW = T // NW
STEPS = TPW // L

BLK = 2048
GPB = BLK // 128


def _logits_body(x_ref, w_ref, b_ref, out_ref):
    lgt = lax.dot_general(
        w_ref[...], x_ref[...],
        (((1,), (1,)), ((), ())),
        preferred_element_type=jnp.float32,
    ) + b_ref[...]
    for g in range(GPB):
        out_ref[pl.ds(g * E, E), :] = lgt[:, g * 128:(g + 1) * 128]


_logits_call = pl.pallas_call(
    _logits_body,
    grid=(T // BLK,),
    in_specs=[
        pl.BlockSpec((BLK, D), lambda i: (i, 0)),
        pl.BlockSpec((E, D), lambda i: (0, 0)),
        pl.BlockSpec((E, 1), lambda i: (0, 0)),
    ],
    out_specs=pl.BlockSpec((GPB * E, 128), lambda i: (i, 0)),
    out_shape=jax.ShapeDtypeStruct((T // 16, 128), jnp.float32),
    compiler_params=pltpu.CompilerParams(
        dimension_semantics=("arbitrary",),
    ),
)


@functools.partial(
    pl.kernel,
    out_type=(
        jax.ShapeDtypeStruct((K * T,), jnp.int32),
        jax.ShapeDtypeStruct((K * T,), jnp.float32),
    ),
    mesh=plsc.VectorSubcoreMesh(core_axis_name="c", subcore_axis_name="s"),
    compiler_params=pltpu.CompilerParams(needs_layout_passes=False),
    scratch_types=[
        pltpu.VMEM((TPW // 128 * E, 128), jnp.float32),
        pltpu.VMEM((TPW,), jnp.int32),
        pltpu.VMEM((TPW,), jnp.int32),
        pltpu.VMEM((TPW,), jnp.float32),
        pltpu.VMEM((TPW,), jnp.float32),
    ],
)
def _route(lg_hbm, idx_hbm, gate_hbm, lg_v, i1_v, i2_v, g1_v, g2_v):
    wid = lax.axis_index("s") * NC + lax.axis_index("c")
    base = wid * TPW
    rows = TPW // 128 * E
    pltpu.sync_copy(lg_hbm.at[pl.ds(wid * rows, rows), :], lg_v)

    def body(i, carry):
        gl = i >> 3
        c0 = (i & 7) * L
        r0 = gl * E
        ls = [lg_v[r0 + e, pl.ds(c0, L)] for e in range(E)]
        v1 = ls[0]
        i1 = jnp.zeros((L,), jnp.int32)
        v2 = jnp.full((L,), -jnp.inf, jnp.float32)
        i2 = jnp.zeros((L,), jnp.int32)
        for e in range(1, E):
            le = ls[e]
            ee = jnp.full((L,), e, jnp.int32)
            gt1 = le > v1
            gt2 = le > v2
            v2 = jnp.where(gt1, v1, jnp.where(gt2, le, v2))
            i2 = jnp.where(gt1, i1, jnp.where(gt2, ee, i2))
            v1 = jnp.where(gt1, le, v1)
            i1 = jnp.where(gt1, ee, i1)
        s = jnp.exp(ls[0] - v1)
        for e in range(1, E):
            s = s + jnp.exp(ls[e] - v1)
        o = i * L
        i1_v[pl.ds(o, L)] = i1
        i2_v[pl.ds(o, L)] = i2
        g1_v[pl.ds(o, L)] = 1.0 / s
        g2_v[pl.ds(o, L)] = jnp.exp(v2 - v1) / s
        return carry

    lax.fori_loop(0, STEPS, body, 0)
    pltpu.sync_copy(i1_v, idx_hbm.at[pl.ds(base, TPW)])
    pltpu.sync_copy(i2_v, idx_hbm.at[pl.ds(T + base, TPW)])
    pltpu.sync_copy(g1_v, gate_hbm.at[pl.ds(base, TPW)])
    pltpu.sync_copy(g2_v, gate_hbm.at[pl.ds(T + base, TPW)])


def kernel(x, W, b):
    lgt = _logits_call(x, W, b.reshape(E, 1))
    idx_f, gate_f = _route(lgt)
    expert_idx = jnp.stack([idx_f[:T], idx_f[T:]], axis=1)
    gate_vals = jnp.stack([gate_f[:T], gate_f[T:]], axis=1)
    return expert_idx, gate_vals
